# baseline (speedup 1.0000x reference)
import functools

import jax
import jax.numpy as jnp
from jax import lax
from jax.experimental import pallas as pl
from jax.experimental.pallas import tpu as pltpu
from jax.experimental.pallas import tpu_sc as plsc

B = 16384
D = 64
NC = 2
NS = 16
NW = NC * NS
BPW = B // NW

CB = 512


def _round_half(v):
    h = (v + 1) // 2
    return ((h + CB - 1) // CB) * CB


def _tc_repack(table_t, half):
    grid = half // CB

    def body(lo_ref, hi_ref, o_ref):
        o_ref[...] = jnp.concatenate([lo_ref[...], hi_ref[...]], axis=0).T

    hi_blocks = half // CB
    assert 2 * half < table_t.shape[1] + CB
    return pl.pallas_call(
        body,
        grid=(grid,),
        in_specs=[
            pl.BlockSpec((D, CB), lambda k: (0, k)),
            pl.BlockSpec((D, CB), lambda k: (0, k + hi_blocks)),
        ],
        out_specs=pl.BlockSpec((CB, 2 * D), lambda k: (k, 0)),
        out_shape=jax.ShapeDtypeStruct((half, 2 * D), jnp.float32),
    )(table_t, table_t)


def _sc_entity_ids(i_ids, item_map):
    mesh = plsc.VectorSubcoreMesh(core_axis_name="c", subcore_axis_name="s")

    @functools.partial(
        pl.kernel,
        mesh=mesh,
        out_type=jax.ShapeDtypeStruct((B,), jnp.int32),
        compiler_params=pltpu.CompilerParams(use_tc_tiling_on_sc=False),
        scratch_types=[
            pltpu.VMEM((BPW,), jnp.int32),
            pltpu.VMEM((BPW,), jnp.int32),
            pltpu.SemaphoreType.DMA,
        ],
    )
    def ka(i_ids_h, map_h, evar_out, iidx, evar_v, s0):
        wid = lax.axis_index("s") * NC + lax.axis_index("c")
        base = wid * BPW
        pltpu.sync_copy(i_ids_h.at[pl.ds(base, BPW)], iidx)
        pltpu.async_copy(map_h.at[iidx], evar_v, s0).wait()
        pltpu.sync_copy(evar_v, evar_out.at[pl.ds(base, BPW)])

    return ka(i_ids, item_map)


def _sc_gather_pairs(u_ids, i_ids, evar, u2, i2, e2, uh_half, ih_half, eh_half):
    mesh = plsc.VectorSubcoreMesh(core_axis_name="c", subcore_axis_name="s")
    out_types = (
        jax.ShapeDtypeStruct((B, 2 * D), jnp.float32),
        jax.ShapeDtypeStruct((B, 2 * D), jnp.float32),
        jax.ShapeDtypeStruct((B, 2 * D), jnp.float32),
    )

    @functools.partial(
        pl.kernel,
        mesh=mesh,
        out_type=out_types,
        scratch_types=[
            pltpu.VMEM((BPW,), jnp.int32),
            pltpu.VMEM((BPW,), jnp.int32),
            pltpu.VMEM((BPW,), jnp.int32),
            pltpu.VMEM((BPW, 2 * D), jnp.float32),
            pltpu.SemaphoreType.DMA,
        ],
    )
    def kb(u_ids_h, i_ids_h, evar_h, u2_h, i2_h, e2_h, u_out, i_out, e_out,
           uh, ih, eh, rows, sem):
        wid = lax.axis_index("s") * NC + lax.axis_index("c")
        base = wid * BPW
        pltpu.sync_copy(u_ids_h.at[pl.ds(base, BPW)], uh)
        pltpu.sync_copy(i_ids_h.at[pl.ds(base, BPW)], ih)
        pltpu.sync_copy(evar_h.at[pl.ds(base, BPW)], eh)

        @pl.loop(0, BPW, step=16)
        def _(c):
            slc = pl.ds(c, 16)
            uv = uh[slc]
            uh[slc] = uv - jnp.where(uv >= uh_half, uh_half, 0)
            iv = ih[slc]
            ih[slc] = iv - jnp.where(iv >= ih_half, ih_half, 0)
            ev = eh[slc]
            eh[slc] = ev - jnp.where(ev >= eh_half, eh_half, 0)

        pltpu.async_copy(u2_h.at[uh], rows, sem).wait()
        pltpu.sync_copy(rows, u_out.at[pl.ds(base, BPW)])
        pltpu.async_copy(i2_h.at[ih], rows, sem).wait()
        pltpu.sync_copy(rows, i_out.at[pl.ds(base, BPW)])
        pltpu.async_copy(e2_h.at[eh], rows, sem).wait()
        pltpu.sync_copy(rows, e_out.at[pl.ds(base, BPW)])

    return kb(u_ids, i_ids, evar, u2, i2, e2)


def _tc_score(u_pair, i_pair, e_pair, u_ids, i_ids, evar, uh, ih, eh):
    def body(u_ref, i_ref, e_ref, uid_ref, iid_ref, eid_ref, o_ref):
        def sel(pair, ids, half):
            return jnp.where(ids >= half, pair[:, D:], pair[:, :D])

        u = sel(u_ref[...], uid_ref[...], uh)
        ie = (sel(i_ref[...], iid_ref[...], ih)
              + sel(e_ref[...], eid_ref[...], eh))
        o_ref[...] = jnp.sum(u * ie, axis=-1)

    blk = 2048
    grid = B // blk
    pair_spec = pl.BlockSpec((blk, 2 * D), lambda i: (i, 0))
    id_spec = pl.BlockSpec((blk, 1), lambda i: (i, 0))
    return pl.pallas_call(
        body,
        grid=(grid,),
        in_specs=[pair_spec, pair_spec, pair_spec, id_spec, id_spec, id_spec],
        out_specs=pl.BlockSpec((blk,), lambda i: (i,)),
        out_shape=jax.ShapeDtypeStruct((B,), jnp.float32),
    )(u_pair, i_pair, e_pair, u_ids.reshape(B, 1), i_ids.reshape(B, 1),
      evar.reshape(B, 1))


def kernel(u_ids, i_ids, user_emb, item_emb, ent_emb, item_map):
    u_ids = u_ids.astype(jnp.int32)
    i_ids = i_ids.astype(jnp.int32)
    U = user_emb.shape[0]
    I = item_emb.shape[0]
    E = ent_emb.shape[0]
    uh, ih, eh = _round_half(U), _round_half(I), _round_half(E)
    u2 = _tc_repack(user_emb.T, uh)
    i2 = _tc_repack(item_emb.T, ih)
    e2 = _tc_repack(ent_emb.T, eh)
    evar = _sc_entity_ids(i_ids, item_map.astype(jnp.int32))
    u_pair, i_pair, e_pair = _sc_gather_pairs(
        u_ids, i_ids, evar, u2, i2, e2, uh, ih, eh)
    return _tc_score(u_pair, i_pair, e_pair, u_ids, i_ids, evar, uh, ih, eh)


# speedup vs baseline: 2.6815x; 2.6815x over previous
import functools

import jax
import jax.numpy as jnp
from jax import lax
from jax.experimental import pallas as pl
from jax.experimental.pallas import tpu as pltpu
from jax.experimental.pallas import tpu_sc as plsc

B = 16384
D = 64
NC = 2
NS = 16
NW = NC * NS
BPW = B // NW

CB = 4096


def _round_half(v):
    h = (v + 1) // 2
    return ((h + CB - 1) // CB) * CB


def _tc_repack(table_t, half):
    grid = half // CB

    def body(lo_ref, hi_ref, o_ref):
        o_ref[...] = jnp.concatenate([lo_ref[...], hi_ref[...]], axis=0).T

    hi_blocks = half // CB
    last_block = (table_t.shape[1] - 1) // CB
    return pl.pallas_call(
        body,
        grid=(grid,),
        in_specs=[
            pl.BlockSpec((D, CB), lambda k: (0, k)),
            pl.BlockSpec((D, CB),
                         lambda k: (0, jnp.minimum(k + hi_blocks, last_block))),
        ],
        out_specs=pl.BlockSpec((CB, 2 * D), lambda k: (k, 0)),
        out_shape=jax.ShapeDtypeStruct((half, 2 * D), jnp.float32),
    )(table_t, table_t)


def _sc_entity_ids(i_ids, item_map):
    mesh = plsc.VectorSubcoreMesh(core_axis_name="c", subcore_axis_name="s")

    @functools.partial(
        pl.kernel,
        mesh=mesh,
        out_type=jax.ShapeDtypeStruct((B,), jnp.int32),
        compiler_params=pltpu.CompilerParams(use_tc_tiling_on_sc=False),
        scratch_types=[
            pltpu.VMEM((BPW,), jnp.int32),
            pltpu.VMEM((BPW,), jnp.int32),
            pltpu.SemaphoreType.DMA,
        ],
    )
    def ka(i_ids_h, map_h, evar_out, iidx, evar_v, s0):
        wid = lax.axis_index("s") * NC + lax.axis_index("c")
        base = wid * BPW
        pltpu.sync_copy(i_ids_h.at[pl.ds(base, BPW)], iidx)
        pltpu.async_copy(map_h.at[iidx], evar_v, s0).wait()
        pltpu.sync_copy(evar_v, evar_out.at[pl.ds(base, BPW)])

    return ka(i_ids, item_map)


def _sc_gather_pairs(u_ids, i_ids, evar, u2, i2, e2, uh_half, ih_half, eh_half):
    mesh = plsc.VectorSubcoreMesh(core_axis_name="c", subcore_axis_name="s")
    out_types = (
        jax.ShapeDtypeStruct((B, 2 * D), jnp.float32),
        jax.ShapeDtypeStruct((B, 2 * D), jnp.float32),
        jax.ShapeDtypeStruct((B, 2 * D), jnp.float32),
    )

    @functools.partial(
        pl.kernel,
        mesh=mesh,
        out_type=out_types,
        scratch_types=[
            pltpu.VMEM((BPW,), jnp.int32),
            pltpu.VMEM((BPW,), jnp.int32),
            pltpu.VMEM((BPW,), jnp.int32),
            pltpu.VMEM((BPW, 2 * D), jnp.float32),
            pltpu.SemaphoreType.DMA,
        ],
    )
    def kb(u_ids_h, i_ids_h, evar_h, u2_h, i2_h, e2_h, u_out, i_out, e_out,
           uh, ih, eh, rows, sem):
        wid = lax.axis_index("s") * NC + lax.axis_index("c")
        base = wid * BPW
        pltpu.sync_copy(u_ids_h.at[pl.ds(base, BPW)], uh)
        pltpu.sync_copy(i_ids_h.at[pl.ds(base, BPW)], ih)
        pltpu.sync_copy(evar_h.at[pl.ds(base, BPW)], eh)

        @pl.loop(0, BPW, step=16)
        def _(c):
            slc = pl.ds(c, 16)
            uv = uh[slc]
            uh[slc] = uv - jnp.where(uv >= uh_half, uh_half, 0)
            iv = ih[slc]
            ih[slc] = iv - jnp.where(iv >= ih_half, ih_half, 0)
            ev = eh[slc]
            eh[slc] = ev - jnp.where(ev >= eh_half, eh_half, 0)

        pltpu.async_copy(u2_h.at[uh], rows, sem).wait()
        pltpu.sync_copy(rows, u_out.at[pl.ds(base, BPW)])
        pltpu.async_copy(i2_h.at[ih], rows, sem).wait()
        pltpu.sync_copy(rows, i_out.at[pl.ds(base, BPW)])
        pltpu.async_copy(e2_h.at[eh], rows, sem).wait()
        pltpu.sync_copy(rows, e_out.at[pl.ds(base, BPW)])

    return kb(u_ids, i_ids, evar, u2, i2, e2)


def _tc_score(u_pair, i_pair, e_pair, u_ids, i_ids, evar, uh, ih, eh):
    def body(u_ref, i_ref, e_ref, uid_ref, iid_ref, eid_ref, o_ref):
        def sel(pair, ids, half):
            return jnp.where(ids >= half, pair[:, D:], pair[:, :D])

        u = sel(u_ref[...], uid_ref[...], uh)
        ie = (sel(i_ref[...], iid_ref[...], ih)
              + sel(e_ref[...], eid_ref[...], eh))
        o_ref[...] = jnp.sum(u * ie, axis=-1)

    blk = 2048
    grid = B // blk
    pair_spec = pl.BlockSpec((blk, 2 * D), lambda i: (i, 0))
    id_spec = pl.BlockSpec((blk, 1), lambda i: (i, 0))
    return pl.pallas_call(
        body,
        grid=(grid,),
        in_specs=[pair_spec, pair_spec, pair_spec, id_spec, id_spec, id_spec],
        out_specs=pl.BlockSpec((blk,), lambda i: (i,)),
        out_shape=jax.ShapeDtypeStruct((B,), jnp.float32),
    )(u_pair, i_pair, e_pair, u_ids.reshape(B, 1), i_ids.reshape(B, 1),
      evar.reshape(B, 1))


def kernel(u_ids, i_ids, user_emb, item_emb, ent_emb, item_map):
    u_ids = u_ids.astype(jnp.int32)
    i_ids = i_ids.astype(jnp.int32)
    U = user_emb.shape[0]
    I = item_emb.shape[0]
    E = ent_emb.shape[0]
    uh, ih, eh = _round_half(U), _round_half(I), _round_half(E)
    u2 = _tc_repack(user_emb.T, uh)
    i2 = _tc_repack(item_emb.T, ih)
    e2 = _tc_repack(ent_emb.T, eh)
    evar = _sc_entity_ids(i_ids, item_map.astype(jnp.int32))
    u_pair, i_pair, e_pair = _sc_gather_pairs(
        u_ids, i_ids, evar, u2, i2, e2, uh, ih, eh)
    return _tc_score(u_pair, i_pair, e_pair, u_ids, i_ids, evar, uh, ih, eh)


# speedup vs baseline: 2.9944x; 1.1167x over previous
import functools

import jax
import jax.numpy as jnp
from jax import lax
from jax.experimental import pallas as pl
from jax.experimental.pallas import tpu as pltpu
from jax.experimental.pallas import tpu_sc as plsc

B = 16384
D = 64
NC = 2
NS = 16
NW = NC * NS
BPW = B // NW

CB = 8192


def _round_half(v):
    h = (v + 1) // 2
    return ((h + CB - 1) // CB) * CB


def _tc_repack(table_t, half):
    grid = half // CB

    def body(lo_ref, hi_ref, o_ref):
        o_ref[...] = jnp.concatenate([lo_ref[...], hi_ref[...]], axis=0).T

    hi_blocks = half // CB
    last_block = (table_t.shape[1] - 1) // CB
    return pl.pallas_call(
        body,
        grid=(grid,),
        in_specs=[
            pl.BlockSpec((D, CB), lambda k: (0, k)),
            pl.BlockSpec((D, CB),
                         lambda k: (0, jnp.minimum(k + hi_blocks, last_block))),
        ],
        out_specs=pl.BlockSpec((CB, 2 * D), lambda k: (k, 0)),
        out_shape=jax.ShapeDtypeStruct((half, 2 * D), jnp.float32),
    )(table_t, table_t)


def _sc_entity_ids(i_ids, item_map):
    mesh = plsc.VectorSubcoreMesh(core_axis_name="c", subcore_axis_name="s")

    @functools.partial(
        pl.kernel,
        mesh=mesh,
        out_type=jax.ShapeDtypeStruct((B,), jnp.int32),
        compiler_params=pltpu.CompilerParams(use_tc_tiling_on_sc=False),
        scratch_types=[
            pltpu.VMEM((BPW,), jnp.int32),
            pltpu.VMEM((BPW,), jnp.int32),
            pltpu.SemaphoreType.DMA,
        ],
    )
    def ka(i_ids_h, map_h, evar_out, iidx, evar_v, s0):
        wid = lax.axis_index("s") * NC + lax.axis_index("c")
        base = wid * BPW
        pltpu.sync_copy(i_ids_h.at[pl.ds(base, BPW)], iidx)
        pltpu.async_copy(map_h.at[iidx], evar_v, s0).wait()
        pltpu.sync_copy(evar_v, evar_out.at[pl.ds(base, BPW)])

    return ka(i_ids, item_map)


def _sc_gather_pairs(u_ids, i_ids, evar, u2, i2, e2, uh_half, ih_half, eh_half):
    mesh = plsc.VectorSubcoreMesh(core_axis_name="c", subcore_axis_name="s")
    out_types = (
        jax.ShapeDtypeStruct((B, 2 * D), jnp.float32),
        jax.ShapeDtypeStruct((B, 2 * D), jnp.float32),
        jax.ShapeDtypeStruct((B, 2 * D), jnp.float32),
    )

    @functools.partial(
        pl.kernel,
        mesh=mesh,
        out_type=out_types,
        scratch_types=[
            pltpu.VMEM((BPW,), jnp.int32),
            pltpu.VMEM((BPW,), jnp.int32),
            pltpu.VMEM((BPW,), jnp.int32),
            pltpu.VMEM((BPW, 2 * D), jnp.float32),
            pltpu.SemaphoreType.DMA,
        ],
    )
    def kb(u_ids_h, i_ids_h, evar_h, u2_h, i2_h, e2_h, u_out, i_out, e_out,
           uh, ih, eh, rows, sem):
        wid = lax.axis_index("s") * NC + lax.axis_index("c")
        base = wid * BPW
        pltpu.sync_copy(u_ids_h.at[pl.ds(base, BPW)], uh)
        pltpu.sync_copy(i_ids_h.at[pl.ds(base, BPW)], ih)
        pltpu.sync_copy(evar_h.at[pl.ds(base, BPW)], eh)

        @pl.loop(0, BPW, step=16)
        def _(c):
            slc = pl.ds(c, 16)
            uv = uh[slc]
            uh[slc] = uv - jnp.where(uv >= uh_half, uh_half, 0)
            iv = ih[slc]
            ih[slc] = iv - jnp.where(iv >= ih_half, ih_half, 0)
            ev = eh[slc]
            eh[slc] = ev - jnp.where(ev >= eh_half, eh_half, 0)

        pltpu.async_copy(u2_h.at[uh], rows, sem).wait()
        pltpu.sync_copy(rows, u_out.at[pl.ds(base, BPW)])
        pltpu.async_copy(i2_h.at[ih], rows, sem).wait()
        pltpu.sync_copy(rows, i_out.at[pl.ds(base, BPW)])
        pltpu.async_copy(e2_h.at[eh], rows, sem).wait()
        pltpu.sync_copy(rows, e_out.at[pl.ds(base, BPW)])

    return kb(u_ids, i_ids, evar, u2, i2, e2)


def _tc_score(u_pair, i_pair, e_pair, ids3, uh, ih, eh):
    def body(u_ref, i_ref, e_ref, ids_ref, o_ref):
        ids = ids_ref[...]

        def sel(pair, col, half):
            return jnp.where(ids[:, col:col + 1] >= half,
                             pair[:, D:], pair[:, :D])

        u = sel(u_ref[...], 0, uh)
        ie = sel(i_ref[...], 1, ih) + sel(e_ref[...], 2, eh)
        o_ref[...] = jnp.sum(u * ie, axis=-1)

    blk = 8192
    grid = B // blk
    pair_spec = pl.BlockSpec((blk, 2 * D), lambda i: (i, 0))
    return pl.pallas_call(
        body,
        grid=(grid,),
        in_specs=[pair_spec, pair_spec, pair_spec,
                  pl.BlockSpec((blk, 3), lambda i: (i, 0))],
        out_specs=pl.BlockSpec((blk,), lambda i: (i,)),
        out_shape=jax.ShapeDtypeStruct((B,), jnp.float32),
    )(u_pair, i_pair, e_pair, ids3)


def kernel(u_ids, i_ids, user_emb, item_emb, ent_emb, item_map):
    u_ids = u_ids.astype(jnp.int32)
    i_ids = i_ids.astype(jnp.int32)
    U = user_emb.shape[0]
    I = item_emb.shape[0]
    E = ent_emb.shape[0]
    uh, ih, eh = _round_half(U), _round_half(I), _round_half(E)
    u2 = _tc_repack(user_emb.T, uh)
    i2 = _tc_repack(item_emb.T, ih)
    e2 = _tc_repack(ent_emb.T, eh)
    evar = _sc_entity_ids(i_ids, item_map.astype(jnp.int32))
    u_pair, i_pair, e_pair = _sc_gather_pairs(
        u_ids, i_ids, evar, u2, i2, e2, uh, ih, eh)
    ids3 = jnp.stack([u_ids, i_ids, evar], axis=1)
    return _tc_score(u_pair, i_pair, e_pair, ids3, uh, ih, eh)


# speedup vs baseline: 3.0175x; 1.0077x over previous
import functools

import jax
import jax.numpy as jnp
from jax import lax
from jax.experimental import pallas as pl
from jax.experimental.pallas import tpu as pltpu
from jax.experimental.pallas import tpu_sc as plsc

B = 16384
D = 64
NC = 2
NS = 16
NW = NC * NS
BPW = B // NW

CB = 8192


def _round_half(v):
    h = (v + 1) // 2
    return ((h + CB - 1) // CB) * CB


def _tc_repack(table_t, half):
    grid = half // CB

    def body(lo_ref, hi_ref, o_ref):
        o_ref[...] = jnp.concatenate([lo_ref[...], hi_ref[...]], axis=0).T

    hi_blocks = half // CB
    last_block = (table_t.shape[1] - 1) // CB
    return pl.pallas_call(
        body,
        grid=(grid,),
        in_specs=[
            pl.BlockSpec((D, CB), lambda k: (0, k)),
            pl.BlockSpec((D, CB),
                         lambda k: (0, jnp.minimum(k + hi_blocks, last_block))),
        ],
        out_specs=pl.BlockSpec((CB, 2 * D), lambda k: (k, 0)),
        out_shape=jax.ShapeDtypeStruct((half, 2 * D), jnp.float32),
    )(table_t, table_t)


def _sc_entity_ids(i_ids, item_map):
    mesh = plsc.VectorSubcoreMesh(core_axis_name="c", subcore_axis_name="s")

    @functools.partial(
        pl.kernel,
        mesh=mesh,
        out_type=jax.ShapeDtypeStruct((B,), jnp.int32),
        compiler_params=pltpu.CompilerParams(use_tc_tiling_on_sc=False),
        scratch_types=[
            pltpu.VMEM((BPW,), jnp.int32),
            pltpu.VMEM((BPW,), jnp.int32),
            pltpu.SemaphoreType.DMA,
        ],
    )
    def ka(i_ids_h, map_h, evar_out, iidx, evar_v, s0):
        wid = lax.axis_index("s") * NC + lax.axis_index("c")
        base = wid * BPW
        pltpu.sync_copy(i_ids_h.at[pl.ds(base, BPW)], iidx)
        pltpu.async_copy(map_h.at[iidx], evar_v, s0).wait()
        pltpu.sync_copy(evar_v, evar_out.at[pl.ds(base, BPW)])

    return ka(i_ids, item_map)


def _sc_gather2(ids_a, ids_b, t2_a, t2_b, half_a, half_b):
    mesh = plsc.VectorSubcoreMesh(core_axis_name="c", subcore_axis_name="s")
    out_types = (
        jax.ShapeDtypeStruct((B, 2 * D), jnp.float32),
        jax.ShapeDtypeStruct((B, 2 * D), jnp.float32),
    )

    @functools.partial(
        pl.kernel,
        mesh=mesh,
        out_type=out_types,
        scratch_types=[
            pltpu.VMEM((BPW,), jnp.int32),
            pltpu.VMEM((BPW,), jnp.int32),
            pltpu.VMEM((BPW, 2 * D), jnp.float32),
            pltpu.SemaphoreType.DMA,
        ],
    )
    def kb(a_ids_h, b_ids_h, a2_h, b2_h, a_out, b_out,
           ah, bh, rows, sa):
        wid = lax.axis_index("s") * NC + lax.axis_index("c")
        base = wid * BPW
        pltpu.sync_copy(a_ids_h.at[pl.ds(base, BPW)], ah)
        pltpu.sync_copy(b_ids_h.at[pl.ds(base, BPW)], bh)

        @pl.loop(0, BPW, step=16)
        def _(c):
            slc = pl.ds(c, 16)
            av = ah[slc]
            ah[slc] = av - jnp.where(av >= half_a, half_a, 0)
            bv = bh[slc]
            bh[slc] = bv - jnp.where(bv >= half_b, half_b, 0)

        pltpu.async_copy(a2_h.at[ah], rows, sa).wait()
        pltpu.sync_copy(rows, a_out.at[pl.ds(base, BPW)])
        pltpu.async_copy(b2_h.at[bh], rows, sa).wait()
        pltpu.sync_copy(rows, b_out.at[pl.ds(base, BPW)])

    return kb(ids_a, ids_b, t2_a, t2_b)


def _sc_gather1(ids_a, t2_a, half_a):
    mesh = plsc.VectorSubcoreMesh(core_axis_name="c", subcore_axis_name="s")

    @functools.partial(
        pl.kernel,
        mesh=mesh,
        out_type=jax.ShapeDtypeStruct((B, 2 * D), jnp.float32),
        scratch_types=[
            pltpu.VMEM((BPW,), jnp.int32),
            pltpu.VMEM((BPW, 2 * D), jnp.float32),
            pltpu.SemaphoreType.DMA,
        ],
    )
    def kb(a_ids_h, a2_h, a_out, ah, rows_a, sa):
        wid = lax.axis_index("s") * NC + lax.axis_index("c")
        base = wid * BPW
        pltpu.sync_copy(a_ids_h.at[pl.ds(base, BPW)], ah)

        @pl.loop(0, BPW, step=16)
        def _(c):
            slc = pl.ds(c, 16)
            av = ah[slc]
            ah[slc] = av - jnp.where(av >= half_a, half_a, 0)

        pltpu.async_copy(a2_h.at[ah], rows_a, sa).wait()
        pltpu.sync_copy(rows_a, a_out.at[pl.ds(base, BPW)])

    return kb(ids_a, t2_a)


def _tc_score(u_pair, i_pair, e_pair, ids3, uh, ih, eh):
    def body(u_ref, i_ref, e_ref, ids_ref, o_ref):
        ids = ids_ref[...]

        def sel(pair, col, half):
            return jnp.where(ids[:, col:col + 1] >= half,
                             pair[:, D:], pair[:, :D])

        u = sel(u_ref[...], 0, uh)
        ie = sel(i_ref[...], 1, ih) + sel(e_ref[...], 2, eh)
        o_ref[...] = jnp.sum(u * ie, axis=-1)

    blk = 8192
    grid = B // blk
    pair_spec = pl.BlockSpec((blk, 2 * D), lambda i: (i, 0))
    return pl.pallas_call(
        body,
        grid=(grid,),
        in_specs=[pair_spec, pair_spec, pair_spec,
                  pl.BlockSpec((blk, 3), lambda i: (i, 0))],
        out_specs=pl.BlockSpec((blk,), lambda i: (i,)),
        out_shape=jax.ShapeDtypeStruct((B,), jnp.float32),
    )(u_pair, i_pair, e_pair, ids3)


def kernel(u_ids, i_ids, user_emb, item_emb, ent_emb, item_map):
    u_ids = u_ids.astype(jnp.int32)
    i_ids = i_ids.astype(jnp.int32)
    U = user_emb.shape[0]
    I = item_emb.shape[0]
    E = ent_emb.shape[0]
    uh, ih, eh = _round_half(U), _round_half(I), _round_half(E)
    evar = _sc_entity_ids(i_ids, item_map.astype(jnp.int32))
    i2 = _tc_repack(item_emb.T, ih)
    e2 = _tc_repack(ent_emb.T, eh)
    i_pair, e_pair = _sc_gather2(i_ids, evar, i2, e2, ih, eh)
    u2 = _tc_repack(user_emb.T, uh)
    u_pair = _sc_gather1(u_ids, u2, uh)
    ids3 = jnp.stack([u_ids, i_ids, evar], axis=1)
    return _tc_score(u_pair, i_pair, e_pair, ids3, uh, ih, eh)


# speedup vs baseline: 3.0536x; 1.0120x over previous
import functools

import jax
import jax.numpy as jnp
from jax import lax
from jax.experimental import pallas as pl
from jax.experimental.pallas import tpu as pltpu
from jax.experimental.pallas import tpu_sc as plsc

B = 16384
D = 64
NC = 2
NS = 16
NW = NC * NS
BPW = B // NW

CB = 8192


def _round_half(v):
    h = (v + 1) // 2
    return ((h + CB - 1) // CB) * CB


def _tc_repack(table_t, half):
    grid = half // CB

    def body(lo_ref, hi_ref, o_ref):
        o_ref[...] = jnp.concatenate([lo_ref[...], hi_ref[...]], axis=0).T

    hi_blocks = half // CB
    last_block = (table_t.shape[1] - 1) // CB
    return pl.pallas_call(
        body,
        grid=(grid,),
        in_specs=[
            pl.BlockSpec((D, CB), lambda k: (0, k)),
            pl.BlockSpec((D, CB),
                         lambda k: (0, jnp.minimum(k + hi_blocks, last_block))),
        ],
        out_specs=pl.BlockSpec((CB, 2 * D), lambda k: (k, 0)),
        out_shape=jax.ShapeDtypeStruct((half, 2 * D), jnp.float32),
    )(table_t, table_t)


def _tc_repack_pair(ta_t, tb_t, half):
    grid = half // CB

    def body(alo_ref, ahi_ref, blo_ref, bhi_ref, oa_ref, ob_ref):
        oa_ref[...] = jnp.concatenate([alo_ref[...], ahi_ref[...]], axis=0).T
        ob_ref[...] = jnp.concatenate([blo_ref[...], bhi_ref[...]], axis=0).T

    hi_blocks = half // CB
    last_a = (ta_t.shape[1] - 1) // CB
    last_b = (tb_t.shape[1] - 1) // CB
    lo_spec = pl.BlockSpec((D, CB), lambda k: (0, k))
    out_spec = pl.BlockSpec((CB, 2 * D), lambda k: (k, 0))
    return pl.pallas_call(
        body,
        grid=(grid,),
        in_specs=[
            lo_spec,
            pl.BlockSpec((D, CB),
                         lambda k: (0, jnp.minimum(k + hi_blocks, last_a))),
            lo_spec,
            pl.BlockSpec((D, CB),
                         lambda k: (0, jnp.minimum(k + hi_blocks, last_b))),
        ],
        out_specs=[out_spec, out_spec],
        out_shape=[jax.ShapeDtypeStruct((half, 2 * D), jnp.float32),
                   jax.ShapeDtypeStruct((half, 2 * D), jnp.float32)],
    )(ta_t, ta_t, tb_t, tb_t)


def _sc_entity_ids(i_ids, item_map):
    mesh = plsc.VectorSubcoreMesh(core_axis_name="c", subcore_axis_name="s")

    @functools.partial(
        pl.kernel,
        mesh=mesh,
        out_type=jax.ShapeDtypeStruct((B,), jnp.int32),
        compiler_params=pltpu.CompilerParams(use_tc_tiling_on_sc=False),
        scratch_types=[
            pltpu.VMEM((BPW,), jnp.int32),
            pltpu.VMEM((BPW,), jnp.int32),
            pltpu.SemaphoreType.DMA,
        ],
    )
    def ka(i_ids_h, map_h, evar_out, iidx, evar_v, s0):
        wid = lax.axis_index("s") * NC + lax.axis_index("c")
        base = wid * BPW
        pltpu.sync_copy(i_ids_h.at[pl.ds(base, BPW)], iidx)
        pltpu.async_copy(map_h.at[iidx], evar_v, s0).wait()
        pltpu.sync_copy(evar_v, evar_out.at[pl.ds(base, BPW)])

    return ka(i_ids, item_map)


def _sc_gather2(ids_a, ids_b, t2_a, t2_b, half_a, half_b):
    mesh = plsc.VectorSubcoreMesh(core_axis_name="c", subcore_axis_name="s")
    out_types = (
        jax.ShapeDtypeStruct((B, 2 * D), jnp.float32),
        jax.ShapeDtypeStruct((B, 2 * D), jnp.float32),
    )

    @functools.partial(
        pl.kernel,
        mesh=mesh,
        out_type=out_types,
        scratch_types=[
            pltpu.VMEM((BPW,), jnp.int32),
            pltpu.VMEM((BPW,), jnp.int32),
            pltpu.VMEM((BPW, 2 * D), jnp.float32),
            pltpu.SemaphoreType.DMA,
        ],
    )
    def kb(a_ids_h, b_ids_h, a2_h, b2_h, a_out, b_out,
           ah, bh, rows, sa):
        wid = lax.axis_index("s") * NC + lax.axis_index("c")
        base = wid * BPW
        pltpu.sync_copy(a_ids_h.at[pl.ds(base, BPW)], ah)
        pltpu.sync_copy(b_ids_h.at[pl.ds(base, BPW)], bh)

        @pl.loop(0, BPW, step=16)
        def _(c):
            slc = pl.ds(c, 16)
            av = ah[slc]
            ah[slc] = av - jnp.where(av >= half_a, half_a, 0)
            bv = bh[slc]
            bh[slc] = bv - jnp.where(bv >= half_b, half_b, 0)

        pltpu.async_copy(a2_h.at[ah], rows, sa).wait()
        pltpu.sync_copy(rows, a_out.at[pl.ds(base, BPW)])
        pltpu.async_copy(b2_h.at[bh], rows, sa).wait()
        pltpu.sync_copy(rows, b_out.at[pl.ds(base, BPW)])

    return kb(ids_a, ids_b, t2_a, t2_b)


def _sc_gather1(ids_a, t2_a, half_a):
    mesh = plsc.VectorSubcoreMesh(core_axis_name="c", subcore_axis_name="s")

    @functools.partial(
        pl.kernel,
        mesh=mesh,
        out_type=jax.ShapeDtypeStruct((B, 2 * D), jnp.float32),
        scratch_types=[
            pltpu.VMEM((BPW,), jnp.int32),
            pltpu.VMEM((BPW, 2 * D), jnp.float32),
            pltpu.SemaphoreType.DMA,
        ],
    )
    def kb(a_ids_h, a2_h, a_out, ah, rows_a, sa):
        wid = lax.axis_index("s") * NC + lax.axis_index("c")
        base = wid * BPW
        pltpu.sync_copy(a_ids_h.at[pl.ds(base, BPW)], ah)

        @pl.loop(0, BPW, step=16)
        def _(c):
            slc = pl.ds(c, 16)
            av = ah[slc]
            ah[slc] = av - jnp.where(av >= half_a, half_a, 0)

        pltpu.async_copy(a2_h.at[ah], rows_a, sa).wait()
        pltpu.sync_copy(rows_a, a_out.at[pl.ds(base, BPW)])

    return kb(ids_a, t2_a)


def _tc_score(u_pair, i_pair, e_pair, ids3, uh, ih, eh):
    def body(u_ref, i_ref, e_ref, ids_ref, o_ref):
        ids = ids_ref[...]

        def sel(pair, col, half):
            return jnp.where(ids[:, col:col + 1] >= half,
                             pair[:, D:], pair[:, :D])

        u = sel(u_ref[...], 0, uh)
        ie = sel(i_ref[...], 1, ih) + sel(e_ref[...], 2, eh)
        o_ref[...] = jnp.sum(u * ie, axis=-1)

    blk = 8192
    grid = B // blk
    pair_spec = pl.BlockSpec((blk, 2 * D), lambda i: (i, 0))
    return pl.pallas_call(
        body,
        grid=(grid,),
        in_specs=[pair_spec, pair_spec, pair_spec,
                  pl.BlockSpec((blk, 3), lambda i: (i, 0))],
        out_specs=pl.BlockSpec((blk,), lambda i: (i,)),
        out_shape=jax.ShapeDtypeStruct((B,), jnp.float32),
    )(u_pair, i_pair, e_pair, ids3)


def kernel(u_ids, i_ids, user_emb, item_emb, ent_emb, item_map):
    u_ids = u_ids.astype(jnp.int32)
    i_ids = i_ids.astype(jnp.int32)
    U = user_emb.shape[0]
    I = item_emb.shape[0]
    E = ent_emb.shape[0]
    uh, ih, eh = _round_half(U), _round_half(I), _round_half(E)
    evar = _sc_entity_ids(i_ids, item_map.astype(jnp.int32))
    if ih == eh:
        i2, e2 = _tc_repack_pair(item_emb.T, ent_emb.T, ih)
    else:
        i2 = _tc_repack(item_emb.T, ih)
        e2 = _tc_repack(ent_emb.T, eh)
    i_pair, e_pair = _sc_gather2(i_ids, evar, i2, e2, ih, eh)
    u2 = _tc_repack(user_emb.T, uh)
    u_pair = _sc_gather1(u_ids, u2, uh)
    ids3 = jnp.stack([u_ids, i_ids, evar], axis=1)
    return _tc_score(u_pair, i_pair, e_pair, ids3, uh, ih, eh)


# speedup vs baseline: 3.1696x; 1.0380x over previous
import functools

import jax
import jax.numpy as jnp
from jax import lax
from jax.experimental import pallas as pl
from jax.experimental.pallas import tpu as pltpu
from jax.experimental.pallas import tpu_sc as plsc

B = 16384
D = 64
NC = 2
NS = 16
NW = NC * NS
BPW = B // NW

CB = 16384
CB2 = 8192


def _round_half(v, cb):
    h = (v + 1) // 2
    return ((h + cb - 1) // cb) * cb


def _tc_repack(table_t, half):
    grid = half // CB

    def body(lo_ref, hi_ref, o_ref):
        o_ref[...] = jnp.concatenate([lo_ref[...], hi_ref[...]], axis=0).T

    hi_blocks = half // CB
    last_block = (table_t.shape[1] - 1) // CB
    return pl.pallas_call(
        body,
        grid=(grid,),
        in_specs=[
            pl.BlockSpec((D, CB), lambda k: (0, k)),
            pl.BlockSpec((D, CB),
                         lambda k: (0, jnp.minimum(k + hi_blocks, last_block))),
        ],
        out_specs=pl.BlockSpec((CB, 2 * D), lambda k: (k, 0)),
        out_shape=jax.ShapeDtypeStruct((half, 2 * D), jnp.float32),
    )(table_t, table_t)


def _tc_repack_pair(ta_t, tb_t, half):
    grid = half // CB2

    def body(alo_ref, ahi_ref, blo_ref, bhi_ref, oa_ref, ob_ref):
        oa_ref[...] = jnp.concatenate([alo_ref[...], ahi_ref[...]], axis=0).T
        ob_ref[...] = jnp.concatenate([blo_ref[...], bhi_ref[...]], axis=0).T

    hi_blocks = half // CB2
    last_a = (ta_t.shape[1] - 1) // CB2
    last_b = (tb_t.shape[1] - 1) // CB2
    lo_spec = pl.BlockSpec((D, CB2), lambda k: (0, k))
    out_spec = pl.BlockSpec((CB2, 2 * D), lambda k: (k, 0))
    return pl.pallas_call(
        body,
        grid=(grid,),
        in_specs=[
            lo_spec,
            pl.BlockSpec((D, CB2),
                         lambda k: (0, jnp.minimum(k + hi_blocks, last_a))),
            lo_spec,
            pl.BlockSpec((D, CB2),
                         lambda k: (0, jnp.minimum(k + hi_blocks, last_b))),
        ],
        out_specs=[out_spec, out_spec],
        out_shape=[jax.ShapeDtypeStruct((half, 2 * D), jnp.float32),
                   jax.ShapeDtypeStruct((half, 2 * D), jnp.float32)],
    )(ta_t, ta_t, tb_t, tb_t)


def _sc_entity_ids(i_ids, item_map):
    mesh = plsc.VectorSubcoreMesh(core_axis_name="c", subcore_axis_name="s")

    @functools.partial(
        pl.kernel,
        mesh=mesh,
        out_type=jax.ShapeDtypeStruct((B,), jnp.int32),
        compiler_params=pltpu.CompilerParams(use_tc_tiling_on_sc=False),
        scratch_types=[
            pltpu.VMEM((BPW,), jnp.int32),
            pltpu.VMEM((BPW,), jnp.int32),
            pltpu.SemaphoreType.DMA,
        ],
    )
    def ka(i_ids_h, map_h, evar_out, iidx, evar_v, s0):
        wid = lax.axis_index("s") * NC + lax.axis_index("c")
        base = wid * BPW
        pltpu.sync_copy(i_ids_h.at[pl.ds(base, BPW)], iidx)
        pltpu.async_copy(map_h.at[iidx], evar_v, s0).wait()
        pltpu.sync_copy(evar_v, evar_out.at[pl.ds(base, BPW)])

    return ka(i_ids, item_map)


def _sc_gather2(ids_a, ids_b, t2_a, t2_b, half_a, half_b):
    mesh = plsc.VectorSubcoreMesh(core_axis_name="c", subcore_axis_name="s")
    out_types = (
        jax.ShapeDtypeStruct((B, 2 * D), jnp.float32),
        jax.ShapeDtypeStruct((B, 2 * D), jnp.float32),
    )

    @functools.partial(
        pl.kernel,
        mesh=mesh,
        out_type=out_types,
        scratch_types=[
            pltpu.VMEM((BPW,), jnp.int32),
            pltpu.VMEM((BPW,), jnp.int32),
            pltpu.VMEM((BPW, 2 * D), jnp.float32),
            pltpu.SemaphoreType.DMA,
        ],
    )
    def kb(a_ids_h, b_ids_h, a2_h, b2_h, a_out, b_out,
           ah, bh, rows, sa):
        wid = lax.axis_index("s") * NC + lax.axis_index("c")
        base = wid * BPW
        pltpu.sync_copy(a_ids_h.at[pl.ds(base, BPW)], ah)
        pltpu.sync_copy(b_ids_h.at[pl.ds(base, BPW)], bh)

        @pl.loop(0, BPW, step=16)
        def _(c):
            slc = pl.ds(c, 16)
            av = ah[slc]
            ah[slc] = av - jnp.where(av >= half_a, half_a, 0)
            bv = bh[slc]
            bh[slc] = bv - jnp.where(bv >= half_b, half_b, 0)

        pltpu.async_copy(a2_h.at[ah], rows, sa).wait()
        pltpu.sync_copy(rows, a_out.at[pl.ds(base, BPW)])
        pltpu.async_copy(b2_h.at[bh], rows, sa).wait()
        pltpu.sync_copy(rows, b_out.at[pl.ds(base, BPW)])

    return kb(ids_a, ids_b, t2_a, t2_b)


def _sc_gather1(ids_a, t2_a, half_a):
    mesh = plsc.VectorSubcoreMesh(core_axis_name="c", subcore_axis_name="s")

    @functools.partial(
        pl.kernel,
        mesh=mesh,
        out_type=jax.ShapeDtypeStruct((B, 2 * D), jnp.float32),
        scratch_types=[
            pltpu.VMEM((BPW,), jnp.int32),
            pltpu.VMEM((BPW, 2 * D), jnp.float32),
            pltpu.SemaphoreType.DMA,
        ],
    )
    def kb(a_ids_h, a2_h, a_out, ah, rows_a, sa):
        wid = lax.axis_index("s") * NC + lax.axis_index("c")
        base = wid * BPW
        pltpu.sync_copy(a_ids_h.at[pl.ds(base, BPW)], ah)

        @pl.loop(0, BPW, step=16)
        def _(c):
            slc = pl.ds(c, 16)
            av = ah[slc]
            ah[slc] = av - jnp.where(av >= half_a, half_a, 0)

        pltpu.async_copy(a2_h.at[ah], rows_a, sa).wait()
        pltpu.sync_copy(rows_a, a_out.at[pl.ds(base, BPW)])

    return kb(ids_a, t2_a)


def _tc_score(u_pair, i_pair, e_pair, ids3, uh, ih, eh):
    def body(u_ref, i_ref, e_ref, ids_ref, o_ref):
        ids = ids_ref[...]

        def sel(pair, col, half):
            return jnp.where(ids[:, col:col + 1] >= half,
                             pair[:, D:], pair[:, :D])

        u = sel(u_ref[...], 0, uh)
        ie = sel(i_ref[...], 1, ih) + sel(e_ref[...], 2, eh)
        o_ref[...] = jnp.sum(u * ie, axis=-1)

    blk = 4096
    grid = B // blk
    pair_spec = pl.BlockSpec((blk, 2 * D), lambda i: (i, 0))
    return pl.pallas_call(
        body,
        grid=(grid,),
        in_specs=[pair_spec, pair_spec, pair_spec,
                  pl.BlockSpec((blk, 3), lambda i: (i, 0))],
        out_specs=pl.BlockSpec((blk,), lambda i: (i,)),
        out_shape=jax.ShapeDtypeStruct((B,), jnp.float32),
    )(u_pair, i_pair, e_pair, ids3)


def kernel(u_ids, i_ids, user_emb, item_emb, ent_emb, item_map):
    u_ids = u_ids.astype(jnp.int32)
    i_ids = i_ids.astype(jnp.int32)
    U = user_emb.shape[0]
    I = item_emb.shape[0]
    E = ent_emb.shape[0]
    uh = _round_half(U, CB)
    ih, eh = _round_half(I, CB2), _round_half(E, CB2)
    evar = _sc_entity_ids(i_ids, item_map.astype(jnp.int32))
    if ih == eh:
        i2, e2 = _tc_repack_pair(item_emb.T, ent_emb.T, ih)
    else:
        i2 = _tc_repack(item_emb.T, ih)
        e2 = _tc_repack(ent_emb.T, eh)
    i_pair, e_pair = _sc_gather2(i_ids, evar, i2, e2, ih, eh)
    u2 = _tc_repack(user_emb.T, uh)
    u_pair = _sc_gather1(u_ids, u2, uh)
    ids3 = jnp.stack([u_ids, i_ids, evar], axis=1)
    return _tc_score(u_pair, i_pair, e_pair, ids3, uh, ih, eh)
